# Initial kernel scaffold; baseline (speedup 1.0000x reference)
import jax
import jax.numpy as jnp
from jax.experimental import pallas as pl


def kernel(x, edge_index, eW0, eb0, eW1, eb1, eW2, eb2, eg, ebt, pW1l, pb1l, pW1r, pW2l, pb2l, pW2r, pg, pbt, dW0, db0, dW1, db1, dW2, db2):
    raise NotImplementedError("write your pallas kernel here")


# speedup vs baseline: 1.0649x; 1.0649x over previous
import functools

import jax
import jax.numpy as jnp
from jax.experimental import pallas as pl
from jax.experimental.pallas import tpu as pltpu

N = 10000
E = 320000
D = 128
LAT = 128
HID = 128
STEPS = 10
OUT = 3

BR = 1000
GRID = N // BR


def _full(shape):
    return pl.BlockSpec(shape, lambda i: (0,) * len(shape))


def _rows(width):
    return pl.BlockSpec((BR, width), lambda i: (i, 0))


def _ln(h, g, b):
    m = jnp.mean(h, axis=-1, keepdims=True)
    v = jnp.mean((h - m) * (h - m), axis=-1, keepdims=True)
    return (h - m) / jnp.sqrt(v + 1e-5) * g + b


def _encode_body(x_ref, w0, b0, w1, b1, w2, b2, g, bt, o_ref):
    h = jnp.maximum(x_ref[...] @ w0[...] + b0[...], 0.0)
    h = jnp.maximum(h @ w1[...] + b1[...], 0.0)
    h = h @ w2[...] + b2[...]
    o_ref[...] = _ln(h, g[...], bt[...])


def _tc_encode(x, w0, b0, w1, b1, w2, b2, g, bt):
    return pl.pallas_call(
        _encode_body,
        grid=(GRID,),
        in_specs=[_rows(D), _full((D, HID)), _full((1, HID)),
                  _full((HID, HID)), _full((1, HID)),
                  _full((HID, LAT)), _full((1, LAT)),
                  _full((1, LAT)), _full((1, LAT))],
        out_specs=_rows(LAT),
        out_shape=jax.ShapeDtypeStruct((N, LAT), jnp.float32),
    )(x, w0, b0, w1, b1, w2, b2, g, bt)


def _comb_relu_body(sp_ref, dp_ref, h_ref, wl, bl, wr, o_ref):
    s = sp_ref[0] + sp_ref[1]
    deg = dp_ref[0][:, 0:1] + dp_ref[1][:, 0:1]
    aggr = s / jnp.maximum(deg, 1.0)
    o_ref[...] = jnp.maximum(aggr @ wl[...] + bl[...] + h_ref[...] @ wr[...], 0.0)


def _tc_comb_relu(sp, dp, h, wl, bl, wr):
    return pl.pallas_call(
        _comb_relu_body,
        grid=(GRID,),
        in_specs=[pl.BlockSpec((2, BR, LAT), lambda i: (0, i, 0)),
                  pl.BlockSpec((2, BR, 16), lambda i: (0, i, 0)),
                  _rows(LAT), _full((LAT, HID)), _full((1, HID)), _full((LAT, HID))],
        out_specs=_rows(HID),
        out_shape=jax.ShapeDtypeStruct((N, HID), jnp.float32),
    )(sp, dp, h, wl, bl, wr)


def _comb_ln_body(sp_ref, dp_ref, h1_ref, hres_ref, wl, bl, wr, g, bt, o_ref):
    s = sp_ref[0] + sp_ref[1]
    deg = dp_ref[0][:, 0:1] + dp_ref[1][:, 0:1]
    aggr = s / jnp.maximum(deg, 1.0)
    h2 = aggr @ wl[...] + bl[...] + h1_ref[...] @ wr[...]
    o_ref[...] = _ln(h2 + hres_ref[...], g[...], bt[...])


def _tc_comb_ln(sp, dp, h1, hres, wl, bl, wr, g, bt):
    return pl.pallas_call(
        _comb_ln_body,
        grid=(GRID,),
        in_specs=[pl.BlockSpec((2, BR, HID), lambda i: (0, i, 0)),
                  pl.BlockSpec((2, BR, 16), lambda i: (0, i, 0)),
                  _rows(HID), _rows(LAT),
                  _full((HID, LAT)), _full((1, LAT)), _full((HID, LAT)),
                  _full((1, LAT)), _full((1, LAT))],
        out_specs=_rows(LAT),
        out_shape=jax.ShapeDtypeStruct((N, LAT), jnp.float32),
    )(sp, dp, h1, hres, wl, bl, wr, g, bt)


def _decode_body(h_ref, w0, b0, w1, b1, w2, b2, o_ref):
    o = jnp.maximum(h_ref[...] @ w0[...] + b0[...], 0.0)
    o = jnp.maximum(o @ w1[...] + b1[...], 0.0)
    o_ref[...] = o @ w2[...] + b2[...]


def _tc_decode(h, w0, b0, w1, b1, w2, b2):
    return pl.pallas_call(
        _decode_body,
        grid=(GRID,),
        in_specs=[_rows(LAT), _full((LAT, HID)), _full((1, HID)),
                  _full((HID, HID)), _full((1, HID)),
                  _full((HID, OUT)), _full((1, OUT))],
        out_specs=_rows(OUT),
        out_shape=jax.ShapeDtypeStruct((N, OUT), jnp.float32),
    )(h, w0, b0, w1, b1, w2, b2)


def _agg(h, src, dst):
    s = jax.ops.segment_sum(h[src], dst, num_segments=N)
    return jnp.stack([s, jnp.zeros_like(s)])


def _deg_partials(dst):
    d = jax.ops.segment_sum(jnp.ones((E,), jnp.float32), dst, num_segments=N)
    d16 = jnp.broadcast_to(d[:, None], (N, 16))
    return jnp.stack([d16, jnp.zeros_like(d16)])


def kernel(x, edge_index, eW0, eb0, eW1, eb1, eW2, eb2, eg, ebt,
           pW1l, pb1l, pW1r, pW2l, pb2l, pW2r, pg, pbt,
           dW0, db0, dW1, db1, dW2, db2):
    src = edge_index[0]
    dst = edge_index[1]
    r = lambda v: v.reshape(1, -1)

    dp = _deg_partials(dst)
    h = _tc_encode(x, eW0, r(eb0), eW1, r(eb1), eW2, r(eb2), r(eg), r(ebt))
    for i in range(STEPS):
        sp = _agg(h, src, dst)
        h1 = _tc_comb_relu(sp, dp, h, pW1l[i], r(pb1l[i]), pW1r[i])
        sp2 = _agg(h1, src, dst)
        h = _tc_comb_ln(sp2, dp, h1, h, pW2l[i], r(pb2l[i]), pW2r[i], r(pg[i]), r(pbt[i]))
    o = _tc_decode(h, dW0, r(db0), dW1, r(db1), dW2, r(db2))
    return o


# speedup vs baseline: 4.2009x; 3.9447x over previous
import functools

import jax
import jax.numpy as jnp
from jax import lax
from jax.experimental import pallas as pl
from jax.experimental.pallas import tpu as pltpu
from jax.experimental.pallas import tpu_sc as plsc

N = 10000
E = 320000
D = 128
LAT = 128
HID = 128
STEPS = 10
OUT = 3

BR = 1000
GRID = N // BR


def _full(shape):
    return pl.BlockSpec(shape, lambda i: (0,) * len(shape))


def _rows(width):
    return pl.BlockSpec((BR, width), lambda i: (i, 0))


def _ln(h, g, b):
    m = jnp.mean(h, axis=-1, keepdims=True)
    v = jnp.mean((h - m) * (h - m), axis=-1, keepdims=True)
    return (h - m) / jnp.sqrt(v + 1e-5) * g + b


def _encode_body(x_ref, w0, b0, w1, b1, w2, b2, g, bt, o_ref):
    h = jnp.maximum(x_ref[...] @ w0[...] + b0[...], 0.0)
    h = jnp.maximum(h @ w1[...] + b1[...], 0.0)
    h = h @ w2[...] + b2[...]
    o_ref[...] = _ln(h, g[...], bt[...])


def _tc_encode(x, w0, b0, w1, b1, w2, b2, g, bt):
    return pl.pallas_call(
        _encode_body,
        grid=(GRID,),
        in_specs=[_rows(D), _full((D, HID)), _full((1, HID)),
                  _full((HID, HID)), _full((1, HID)),
                  _full((HID, LAT)), _full((1, LAT)),
                  _full((1, LAT)), _full((1, LAT))],
        out_specs=_rows(LAT),
        out_shape=jax.ShapeDtypeStruct((N, LAT), jnp.float32),
    )(x, w0, b0, w1, b1, w2, b2, g, bt)


def _comb_relu_body(sp_ref, dp_ref, h_ref, wl, bl, wr, o_ref):
    s = sp_ref[0] + sp_ref[1]
    deg = dp_ref[0][:, 0:1] + dp_ref[1][:, 0:1]
    aggr = s / jnp.maximum(deg, 1.0)
    o_ref[...] = jnp.maximum(aggr @ wl[...] + bl[...] + h_ref[...] @ wr[...], 0.0)


def _tc_comb_relu(sp, dp, h, wl, bl, wr):
    return pl.pallas_call(
        _comb_relu_body,
        grid=(GRID,),
        in_specs=[pl.BlockSpec((2, BR, LAT), lambda i: (0, i, 0)),
                  pl.BlockSpec((2, BR, 16), lambda i: (0, i, 0)),
                  _rows(LAT), _full((LAT, HID)), _full((1, HID)), _full((LAT, HID))],
        out_specs=_rows(HID),
        out_shape=jax.ShapeDtypeStruct((N, HID), jnp.float32),
    )(sp, dp, h, wl, bl, wr)


def _comb_ln_body(sp_ref, dp_ref, h1_ref, hres_ref, wl, bl, wr, g, bt, o_ref):
    s = sp_ref[0] + sp_ref[1]
    deg = dp_ref[0][:, 0:1] + dp_ref[1][:, 0:1]
    aggr = s / jnp.maximum(deg, 1.0)
    h2 = aggr @ wl[...] + bl[...] + h1_ref[...] @ wr[...]
    o_ref[...] = _ln(h2 + hres_ref[...], g[...], bt[...])


def _tc_comb_ln(sp, dp, h1, hres, wl, bl, wr, g, bt):
    return pl.pallas_call(
        _comb_ln_body,
        grid=(GRID,),
        in_specs=[pl.BlockSpec((2, BR, HID), lambda i: (0, i, 0)),
                  pl.BlockSpec((2, BR, 16), lambda i: (0, i, 0)),
                  _rows(HID), _rows(LAT),
                  _full((HID, LAT)), _full((1, LAT)), _full((HID, LAT)),
                  _full((1, LAT)), _full((1, LAT))],
        out_specs=_rows(LAT),
        out_shape=jax.ShapeDtypeStruct((N, LAT), jnp.float32),
    )(sp, dp, h1, hres, wl, bl, wr, g, bt)


def _decode_body(h_ref, w0, b0, w1, b1, w2, b2, o_ref):
    o = jnp.maximum(h_ref[...] @ w0[...] + b0[...], 0.0)
    o = jnp.maximum(o @ w1[...] + b1[...], 0.0)
    o_ref[...] = o @ w2[...] + b2[...]


def _tc_decode(h, w0, b0, w1, b1, w2, b2):
    return pl.pallas_call(
        _decode_body,
        grid=(GRID,),
        in_specs=[_rows(LAT), _full((LAT, HID)), _full((1, HID)),
                  _full((HID, HID)), _full((1, HID)),
                  _full((HID, OUT)), _full((1, OUT))],
        out_specs=_rows(OUT),
        out_shape=jax.ShapeDtypeStruct((N, OUT), jnp.float32),
    )(h, w0, b0, w1, b1, w2, b2)


_NC = 2
_NS = 16
_NW = _NC * _NS
_EPW = E // _NW
_C = 80
_NCHUNK = _EPW // _C
_NPAD = 10240
_RPS = _NPAD // _NS

_sc_mesh = plsc.VectorSubcoreMesh(core_axis_name="c", subcore_axis_name="s")


@functools.partial(
    pl.kernel,
    mesh=_sc_mesh,
    out_type=jax.ShapeDtypeStruct((_NC, _NPAD, LAT), jnp.float32),
    scratch_types=[
        pltpu.VMEM((_C,), jnp.int32),
        pltpu.VMEM((_C,), jnp.int32),
        pltpu.VMEM((_C, LAT), jnp.float32),
        pltpu.VMEM_SHARED((_NPAD, LAT), jnp.float32),
        pltpu.SemaphoreType.DMA,
    ],
)
def _sc_agg_kernel(h_hbm, src_hbm, dst_hbm, zeros_hbm, out_hbm,
                   sidx, didx, rows, acc, sem):
    cid = lax.axis_index("c")
    sid = lax.axis_index("s")
    wid = cid * _NS + sid
    pltpu.sync_copy(zeros_hbm.at[pl.ds(sid * _RPS, _RPS)],
                    acc.at[pl.ds(sid * _RPS, _RPS)])
    plsc.subcore_barrier()
    base = wid * _EPW

    def body(k, carry):
        off = base + k * _C
        pltpu.sync_copy(src_hbm.at[pl.ds(off, _C)], sidx)
        pltpu.sync_copy(dst_hbm.at[pl.ds(off, _C)], didx)
        pltpu.async_copy(h_hbm.at[sidx], rows, sem).wait()
        pltpu.sync_copy(rows, acc.at[didx], add=True)
        return carry

    lax.fori_loop(0, _NCHUNK, body, 0)
    plsc.subcore_barrier()
    pltpu.sync_copy(acc.at[pl.ds(sid * _RPS, _RPS)],
                    out_hbm.at[cid, pl.ds(sid * _RPS, _RPS)])


def _agg(h, src, dst, zeros):
    return _sc_agg_kernel(h, src, dst, zeros)


def _deg_partials(src, dst, zeros):
    ones = jnp.ones((N, LAT), jnp.float32)
    return _sc_agg_kernel(ones, src, dst, zeros)[:, :, :16]


def kernel(x, edge_index, eW0, eb0, eW1, eb1, eW2, eb2, eg, ebt,
           pW1l, pb1l, pW1r, pW2l, pb2l, pW2r, pg, pbt,
           dW0, db0, dW1, db1, dW2, db2):
    src = edge_index[0]
    dst = edge_index[1]
    r = lambda v: v.reshape(1, -1)

    zeros = jnp.zeros((_NPAD, LAT), jnp.float32)
    dp = _deg_partials(src, dst, zeros)
    h = _tc_encode(x, eW0, r(eb0), eW1, r(eb1), eW2, r(eb2), r(eg), r(ebt))
    for i in range(STEPS):
        sp = _agg(h, src, dst, zeros)
        h1 = _tc_comb_relu(sp, dp, h, pW1l[i], r(pb1l[i]), pW1r[i])
        sp2 = _agg(h1, src, dst, zeros)
        h = _tc_comb_ln(sp2, dp, h1, h, pW2l[i], r(pb2l[i]), pW2r[i], r(pg[i]), r(pbt[i]))
    o = _tc_decode(h, dW0, r(db0), dW1, r(db1), dW2, r(db2))
    return o


# speedup vs baseline: 5.5653x; 1.3248x over previous
import functools

import jax
import jax.numpy as jnp
from jax import lax
from jax.experimental import pallas as pl
from jax.experimental.pallas import tpu as pltpu
from jax.experimental.pallas import tpu_sc as plsc

N = 10000
E = 320000
D = 128
LAT = 128
HID = 128
STEPS = 10
OUT = 3
HALF = LAT // 2

BR = 1000
GRID = N // BR


def _full(shape):
    return pl.BlockSpec(shape, lambda i: (0,) * len(shape))


def _rows(width):
    return pl.BlockSpec((BR, width), lambda i: (i, 0))


def _split_rows():
    return pl.BlockSpec((2, BR, HALF), lambda i: (0, i, 0))


def _ln(h, g, b):
    m = jnp.mean(h, axis=-1, keepdims=True)
    v = jnp.mean((h - m) * (h - m), axis=-1, keepdims=True)
    return (h - m) / jnp.sqrt(v + 1e-5) * g + b


def _cat(ref):
    return jnp.concatenate([ref[0], ref[1]], axis=-1)


def _store_split(o_ref, y):
    o_ref[0] = y[:, :HALF]
    o_ref[1] = y[:, HALF:]


def _encode_body(x_ref, w0, b0, w1, b1, w2, b2, g, bt, o_ref):
    h = jnp.maximum(x_ref[...] @ w0[...] + b0[...], 0.0)
    h = jnp.maximum(h @ w1[...] + b1[...], 0.0)
    h = h @ w2[...] + b2[...]
    _store_split(o_ref, _ln(h, g[...], bt[...]))


def _tc_encode(x, w0, b0, w1, b1, w2, b2, g, bt):
    return pl.pallas_call(
        _encode_body,
        grid=(GRID,),
        in_specs=[_rows(D), _full((D, HID)), _full((1, HID)),
                  _full((HID, HID)), _full((1, HID)),
                  _full((HID, LAT)), _full((1, LAT)),
                  _full((1, LAT)), _full((1, LAT))],
        out_specs=_split_rows(),
        out_shape=jax.ShapeDtypeStruct((2, N, HALF), jnp.float32),
    )(x, w0, b0, w1, b1, w2, b2, g, bt)


def _mm_body(h_ref, wr, o_ref):
    o_ref[...] = _cat(h_ref) @ wr[...]


def _tc_mm(h, wr):
    return pl.pallas_call(
        _mm_body,
        grid=(GRID,),
        in_specs=[_split_rows(), _full((LAT, HID))],
        out_specs=_rows(HID),
        out_shape=jax.ShapeDtypeStruct((N, HID), jnp.float32),
    )(h, wr)


def _comb_relu_body(sp_ref, dp_ref, hr_ref, wl, bl, o_ref):
    s = _cat(sp_ref)
    deg = dp_ref[0][:, 0:1]
    aggr = s / jnp.maximum(deg, 1.0)
    _store_split(o_ref, jnp.maximum(aggr @ wl[...] + bl[...] + hr_ref[...], 0.0))


def _tc_comb_relu(sp, dp, hr, wl, bl):
    return pl.pallas_call(
        _comb_relu_body,
        grid=(GRID,),
        in_specs=[_split_rows(),
                  pl.BlockSpec((2, BR, 16), lambda i: (0, i, 0)),
                  _rows(HID), _full((LAT, HID)), _full((1, HID))],
        out_specs=_split_rows(),
        out_shape=jax.ShapeDtypeStruct((2, N, HALF), jnp.float32),
    )(sp, dp, hr, wl, bl)


def _comb_ln_body(sp_ref, dp_ref, hr_ref, hres_ref, wl, bl, g, bt, o_ref):
    s = _cat(sp_ref)
    deg = dp_ref[0][:, 0:1]
    aggr = s / jnp.maximum(deg, 1.0)
    h2 = aggr @ wl[...] + bl[...] + hr_ref[...]
    _store_split(o_ref, _ln(h2 + _cat(hres_ref), g[...], bt[...]))


def _tc_comb_ln(sp, dp, hr, hres, wl, bl, g, bt):
    return pl.pallas_call(
        _comb_ln_body,
        grid=(GRID,),
        in_specs=[_split_rows(),
                  pl.BlockSpec((2, BR, 16), lambda i: (0, i, 0)),
                  _rows(HID), _split_rows(),
                  _full((HID, LAT)), _full((1, LAT)),
                  _full((1, LAT)), _full((1, LAT))],
        out_specs=_split_rows(),
        out_shape=jax.ShapeDtypeStruct((2, N, HALF), jnp.float32),
    )(sp, dp, hr, hres, wl, bl, g, bt)


def _decode_body(h_ref, w0, b0, w1, b1, w2, b2, o_ref):
    o = jnp.maximum(_cat(h_ref) @ w0[...] + b0[...], 0.0)
    o = jnp.maximum(o @ w1[...] + b1[...], 0.0)
    o_ref[...] = o @ w2[...] + b2[...]


def _tc_decode(h, w0, b0, w1, b1, w2, b2):
    return pl.pallas_call(
        _decode_body,
        grid=(GRID,),
        in_specs=[_split_rows(), _full((LAT, HID)), _full((1, HID)),
                  _full((HID, HID)), _full((1, HID)),
                  _full((HID, OUT)), _full((1, OUT))],
        out_specs=_rows(OUT),
        out_shape=jax.ShapeDtypeStruct((N, OUT), jnp.float32),
    )(h, w0, b0, w1, b1, w2, b2)


_NC = 2
_NS = 16
_C = 80
_NCHUNK = 253
_EPT = _NCHUNK * _C
_EPAD = _EPT * _NS
_NPAD = 10240
_RPS = _NPAD // _NS
_NBUF = 4

_sc_mesh = plsc.VectorSubcoreMesh(core_axis_name="c", subcore_axis_name="s")


@functools.partial(
    pl.kernel,
    mesh=_sc_mesh,
    compiler_params=pltpu.CompilerParams(use_tc_tiling_on_sc=False),
    out_type=jax.ShapeDtypeStruct((_NC, _NPAD, HALF), jnp.float32),
    scratch_types=[
        pltpu.VMEM((_NBUF, _C), jnp.int32),
        pltpu.VMEM((_NBUF, _C), jnp.int32),
        pltpu.VMEM((_NBUF, _C, HALF), jnp.float32),
        pltpu.VMEM_SHARED((_NPAD, HALF), jnp.float32),
        pltpu.SemaphoreType.DMA((_NBUF,)),
        pltpu.SemaphoreType.DMA((_NBUF,)),
        pltpu.SemaphoreType.DMA((_NBUF,)),
    ],
)
def _sc_agg_kernel(h_hbm, src_hbm, dst_hbm, zeros_hbm, out_hbm,
                   sidx, didx, rows, acc, sem_i, sem_g, sem_s):
    cid = lax.axis_index("c")
    sid = lax.axis_index("s")
    pltpu.sync_copy(zeros_hbm.at[pl.ds(sid * _RPS, _RPS)],
                    acc.at[pl.ds(sid * _RPS, _RPS)])
    plsc.subcore_barrier()
    base = sid * _EPT
    hc = h_hbm.at[cid]

    def start_idx(off, b):
        pltpu.async_copy(src_hbm.at[pl.ds(off, _C)], sidx.at[b], sem_i.at[b])
        pltpu.async_copy(dst_hbm.at[pl.ds(off, _C)], didx.at[b], sem_i.at[b])

    def wait_idx(b):
        pltpu.make_async_copy(src_hbm.at[pl.ds(0, _C)], sidx.at[b], sem_i.at[b]).wait()
        pltpu.make_async_copy(src_hbm.at[pl.ds(0, _C)], didx.at[b], sem_i.at[b]).wait()

    def start_gather(b):
        pltpu.async_copy(hc.at[sidx.at[b]], rows.at[b], sem_g.at[b])

    def wait_gather(b):
        pltpu.make_async_copy(hc.at[pl.ds(0, _C)], rows.at[b], sem_g.at[b]).wait()

    def start_scatter(b):
        pltpu.async_copy(rows.at[b], acc.at[didx.at[b]], sem_s.at[b], add=True)

    def wait_scatter(b):
        pltpu.make_async_copy(hc.at[pl.ds(0, _C)], rows.at[b], sem_s.at[b]).wait()

    start_idx(base, 0)
    wait_idx(0)
    start_gather(0)
    start_idx(base + _C, 1)
    wait_idx(1)
    start_gather(1)
    start_idx(base + 2 * _C, 2)
    wait_gather(0)
    start_scatter(0)
    wait_idx(2)
    start_gather(2)
    start_idx(base + 3 * _C, 3)
    wait_gather(1)
    start_scatter(1)

    def body(j, carry):
        c0 = 3 + 4 * j
        for i in range(4):
            b = (3 + i) % _NBUF
            pb = (2 + i) % _NBUF
            nb = (4 + i) % _NBUF
            wait_scatter(nb)
            start_idx(base + (c0 + i + 1) * _C, nb)
            wait_idx(b)
            start_gather(b)
            wait_gather(pb)
            start_scatter(pb)
        return carry

    lax.fori_loop(0, (_NCHUNK - 5) // 4, body, 0)

    wait_scatter(0)
    start_idx(base + (_NCHUNK - 1) * _C, 0)
    wait_idx(3)
    start_gather(3)
    wait_gather(2)
    start_scatter(2)
    wait_idx(0)
    start_gather(0)
    wait_gather(3)
    start_scatter(3)
    wait_gather(0)
    start_scatter(0)
    wait_scatter(1)
    wait_scatter(2)
    wait_scatter(3)
    wait_scatter(0)

    plsc.subcore_barrier()
    pltpu.sync_copy(acc.at[pl.ds(sid * _RPS, _RPS)],
                    out_hbm.at[cid, pl.ds(sid * _RPS, _RPS)])


def _agg(h, srcp, dstp, zeros):
    return _sc_agg_kernel(h, srcp, dstp, zeros)


def _deg_partials(srcp, dstp, zeros):
    ones = jnp.ones((2, N, HALF), jnp.float32)
    return _sc_agg_kernel(ones, srcp, dstp, zeros)[:, :, :16]


def kernel(x, edge_index, eW0, eb0, eW1, eb1, eW2, eb2, eg, ebt,
           pW1l, pb1l, pW1r, pW2l, pb2l, pW2r, pg, pbt,
           dW0, db0, dW1, db1, dW2, db2):
    src = edge_index[0]
    dst = edge_index[1]
    pad = _EPAD - E
    srcp = jnp.concatenate([src, jnp.zeros((pad,), jnp.int32)])
    dstp = jnp.concatenate([dst, jnp.full((pad,), _NPAD - 1, jnp.int32)])
    r = lambda v: v.reshape(1, -1)

    zeros = jnp.zeros((_NPAD, HALF), jnp.float32)
    dp = _deg_partials(srcp, dstp, zeros)
    h = _tc_encode(x, eW0, r(eb0), eW1, r(eb1), eW2, r(eb2), r(eg), r(ebt))
    for i in range(STEPS):
        sp = _agg(h, srcp, dstp, zeros)
        hr = _tc_mm(h, pW1r[i])
        h1 = _tc_comb_relu(sp, dp, hr, pW1l[i], r(pb1l[i]))
        sp2 = _agg(h1, srcp, dstp, zeros)
        hr2 = _tc_mm(h1, pW2r[i])
        h = _tc_comb_ln(sp2, dp, hr2, h, pW2l[i], r(pb2l[i]), r(pg[i]), r(pbt[i]))
    o = _tc_decode(h, dW0, r(db0), dW1, r(db1), dW2, r(db2))
    return o


# speedup vs baseline: 11.7590x; 2.1129x over previous
import functools

import jax
import jax.numpy as jnp
from jax import lax
from jax.experimental import pallas as pl
from jax.experimental.pallas import tpu as pltpu
from jax.experimental.pallas import tpu_sc as plsc

N = 10000
E = 320000
D = 128
LAT = 128
HID = 128
STEPS = 10
OUT = 3

BR = 1000
GRID = N // BR


def _full(shape):
    return pl.BlockSpec(shape, lambda i: (0,) * len(shape))


def _rows(width):
    return pl.BlockSpec((BR, width), lambda i: (i, 0))


def _ln(h, g, b):
    m = jnp.mean(h, axis=-1, keepdims=True)
    v = jnp.mean((h - m) * (h - m), axis=-1, keepdims=True)
    return (h - m) / jnp.sqrt(v + 1e-5) * g + b


def _encode_body(x_ref, w0, b0, w1, b1, w2, b2, g, bt, o_ref):
    h = jnp.maximum(x_ref[...] @ w0[...] + b0[...], 0.0)
    h = jnp.maximum(h @ w1[...] + b1[...], 0.0)
    h = h @ w2[...] + b2[...]
    o_ref[...] = _ln(h, g[...], bt[...])


def _tc_encode(x, w0, b0, w1, b1, w2, b2, g, bt):
    return pl.pallas_call(
        _encode_body,
        grid=(GRID,),
        in_specs=[_rows(D), _full((D, HID)), _full((1, HID)),
                  _full((HID, HID)), _full((1, HID)),
                  _full((HID, LAT)), _full((1, LAT)),
                  _full((1, LAT)), _full((1, LAT))],
        out_specs=_rows(LAT),
        out_shape=jax.ShapeDtypeStruct((N, LAT), jnp.float32),
    )(x, w0, b0, w1, b1, w2, b2, g, bt)


def _mm_body(h_ref, wr, o_ref):
    o_ref[...] = h_ref[...] @ wr[...]


def _tc_mm(h, wr):
    return pl.pallas_call(
        _mm_body,
        grid=(GRID,),
        in_specs=[_rows(LAT), _full((LAT, HID))],
        out_specs=_rows(HID),
        out_shape=jax.ShapeDtypeStruct((N, HID), jnp.float32),
    )(h, wr)


def _comb_relu_body(sp_ref, dp_ref, hr_ref, wl, bl, o_ref):
    s = sp_ref[0] + sp_ref[1]
    deg = dp_ref[0][:, 0:1] + dp_ref[1][:, 0:1]
    aggr = s / jnp.maximum(deg, 1.0)
    o_ref[...] = jnp.maximum(aggr @ wl[...] + bl[...] + hr_ref[...], 0.0)


def _tc_comb_relu(sp, dp, hr, wl, bl):
    return pl.pallas_call(
        _comb_relu_body,
        grid=(GRID,),
        in_specs=[pl.BlockSpec((2, BR, LAT), lambda i: (0, i, 0)),
                  pl.BlockSpec((2, BR, 16), lambda i: (0, i, 0)),
                  _rows(HID), _full((LAT, HID)), _full((1, HID))],
        out_specs=_rows(HID),
        out_shape=jax.ShapeDtypeStruct((N, HID), jnp.float32),
    )(sp, dp, hr, wl, bl)


def _comb_ln_body(sp_ref, dp_ref, hr_ref, hres_ref, wl, bl, g, bt, o_ref):
    s = sp_ref[0] + sp_ref[1]
    deg = dp_ref[0][:, 0:1] + dp_ref[1][:, 0:1]
    aggr = s / jnp.maximum(deg, 1.0)
    h2 = aggr @ wl[...] + bl[...] + hr_ref[...]
    o_ref[...] = _ln(h2 + hres_ref[...], g[...], bt[...])


def _tc_comb_ln(sp, dp, hr, hres, wl, bl, g, bt):
    return pl.pallas_call(
        _comb_ln_body,
        grid=(GRID,),
        in_specs=[pl.BlockSpec((2, BR, HID), lambda i: (0, i, 0)),
                  pl.BlockSpec((2, BR, 16), lambda i: (0, i, 0)),
                  _rows(HID), _rows(LAT),
                  _full((HID, LAT)), _full((1, LAT)),
                  _full((1, LAT)), _full((1, LAT))],
        out_specs=_rows(LAT),
        out_shape=jax.ShapeDtypeStruct((N, LAT), jnp.float32),
    )(sp, dp, hr, hres, wl, bl, g, bt)


def _decode_body(h_ref, w0, b0, w1, b1, w2, b2, o_ref):
    o = jnp.maximum(h_ref[...] @ w0[...] + b0[...], 0.0)
    o = jnp.maximum(o @ w1[...] + b1[...], 0.0)
    o_ref[...] = o @ w2[...] + b2[...]


def _tc_decode(h, w0, b0, w1, b1, w2, b2):
    return pl.pallas_call(
        _decode_body,
        grid=(GRID,),
        in_specs=[_rows(LAT), _full((LAT, HID)), _full((1, HID)),
                  _full((HID, HID)), _full((1, HID)),
                  _full((HID, OUT)), _full((1, OUT))],
        out_specs=_rows(OUT),
        out_shape=jax.ShapeDtypeStruct((N, OUT), jnp.float32),
    )(h, w0, b0, w1, b1, w2, b2)


_NC = 2
_NS = 16
_NW = _NC * _NS
_EPW = E // _NW
_C = 80
_NCHUNK = _EPW // _C
_NPAD = 10240
_RPS = _NPAD // _NS

_sc_mesh = plsc.VectorSubcoreMesh(core_axis_name="c", subcore_axis_name="s")


_NBUF = 4


@functools.partial(
    pl.kernel,
    mesh=_sc_mesh,
    out_type=jax.ShapeDtypeStruct((_NC, _NPAD, LAT), jnp.float32),
    scratch_types=[
        pltpu.VMEM((_NBUF, _C), jnp.int32),
        pltpu.VMEM((_NBUF, _C), jnp.int32),
        pltpu.VMEM((_NBUF, _C, LAT), jnp.float32),
        pltpu.VMEM_SHARED((_NPAD, LAT), jnp.float32),
        pltpu.SemaphoreType.DMA((_NBUF,)),
        pltpu.SemaphoreType.DMA((_NBUF,)),
        pltpu.SemaphoreType.DMA((_NBUF,)),
    ],
)
def _sc_agg_kernel(h_hbm, src_hbm, dst_hbm, zeros_hbm, out_hbm,
                   sidx, didx, rows, acc, sem_i, sem_g, sem_s):
    cid = lax.axis_index("c")
    sid = lax.axis_index("s")
    wid = cid * _NS + sid
    pltpu.sync_copy(zeros_hbm.at[pl.ds(sid * _RPS, _RPS)],
                    acc.at[pl.ds(sid * _RPS, _RPS)])
    plsc.subcore_barrier()
    base = wid * _EPW

    def start_idx(off, b):
        pltpu.async_copy(src_hbm.at[pl.ds(off, _C)], sidx.at[b], sem_i.at[b])
        pltpu.async_copy(dst_hbm.at[pl.ds(off, _C)], didx.at[b], sem_i.at[b])

    def wait_idx(b):
        pltpu.make_async_copy(src_hbm.at[pl.ds(0, _C)], sidx.at[b], sem_i.at[b]).wait()
        pltpu.make_async_copy(src_hbm.at[pl.ds(0, _C)], didx.at[b], sem_i.at[b]).wait()

    def start_gather(b):
        pltpu.async_copy(h_hbm.at[sidx.at[b]], rows.at[b], sem_g.at[b])

    def wait_gather(b):
        pltpu.make_async_copy(h_hbm.at[pl.ds(0, _C)], rows.at[b], sem_g.at[b]).wait()

    def start_scatter(b):
        pltpu.async_copy(rows.at[b], acc.at[didx.at[b]], sem_s.at[b], add=True)

    def wait_scatter(b):
        pltpu.make_async_copy(h_hbm.at[pl.ds(0, _C)], rows.at[b], sem_s.at[b]).wait()

    start_idx(base, 0)
    wait_idx(0)
    start_gather(0)
    start_idx(base + _C, 1)
    wait_idx(1)
    start_gather(1)
    start_idx(base + 2 * _C, 2)
    wait_gather(0)
    start_scatter(0)
    wait_idx(2)
    start_gather(2)
    start_idx(base + 3 * _C, 3)
    wait_gather(1)
    start_scatter(1)

    def body(j, carry):
        c0 = 3 + 4 * j
        for i in range(4):
            b = (3 + i) % _NBUF
            pb = (2 + i) % _NBUF
            nb = (4 + i) % _NBUF
            wait_scatter(nb)
            start_idx(base + (c0 + i + 1) * _C, nb)
            wait_idx(b)
            start_gather(b)
            wait_gather(pb)
            start_scatter(pb)
        return carry

    lax.fori_loop(0, (_NCHUNK - 5) // 4, body, 0)

    wait_scatter(0)
    start_idx(base + 124 * _C, 0)
    wait_idx(3)
    start_gather(3)
    wait_gather(2)
    start_scatter(2)
    wait_idx(0)
    start_gather(0)
    wait_gather(3)
    start_scatter(3)
    wait_gather(0)
    start_scatter(0)
    wait_scatter(1)
    wait_scatter(2)
    wait_scatter(3)
    wait_scatter(0)

    plsc.subcore_barrier()
    pltpu.sync_copy(acc.at[pl.ds(sid * _RPS, _RPS)],
                    out_hbm.at[cid, pl.ds(sid * _RPS, _RPS)])


def _agg(h, src, dst, zeros):
    return _sc_agg_kernel(h, src, dst, zeros)


def _deg_partials(src, dst, zeros):
    ones = jnp.ones((N, LAT), jnp.float32)
    return _sc_agg_kernel(ones, src, dst, zeros)[:, :, :16]


def kernel(x, edge_index, eW0, eb0, eW1, eb1, eW2, eb2, eg, ebt,
           pW1l, pb1l, pW1r, pW2l, pb2l, pW2r, pg, pbt,
           dW0, db0, dW1, db1, dW2, db2):
    src = edge_index[0]
    dst = edge_index[1]
    r = lambda v: v.reshape(1, -1)

    zeros = jnp.zeros((_NPAD, LAT), jnp.float32)
    dp = _deg_partials(src, dst, zeros)
    h = _tc_encode(x, eW0, r(eb0), eW1, r(eb1), eW2, r(eb2), r(eg), r(ebt))
    for i in range(STEPS):
        sp = _agg(h, src, dst, zeros)
        hr = _tc_mm(h, pW1r[i])
        h1 = _tc_comb_relu(sp, dp, hr, pW1l[i], r(pb1l[i]))
        sp2 = _agg(h1, src, dst, zeros)
        hr2 = _tc_mm(h1, pW2r[i])
        h = _tc_comb_ln(sp2, dp, hr2, h, pW2l[i], r(pb2l[i]), r(pg[i]), r(pbt[i]))
    o = _tc_decode(h, dW0, r(db0), dW1, r(db1), dW2, r(db2))
    return o


# speedup vs baseline: 11.9881x; 1.0195x over previous
import functools

import jax
import jax.numpy as jnp
from jax import lax
from jax.experimental import pallas as pl
from jax.experimental.pallas import tpu as pltpu
from jax.experimental.pallas import tpu_sc as plsc

N = 10000
E = 320000
D = 128
LAT = 128
HID = 128
STEPS = 10
OUT = 3

BR = 2000
GRID = N // BR


def _full(shape):
    return pl.BlockSpec(shape, lambda i: (0,) * len(shape))


def _rows(width):
    return pl.BlockSpec((BR, width), lambda i: (i, 0))


def _ln(h, g, b):
    m = jnp.mean(h, axis=-1, keepdims=True)
    v = jnp.mean((h - m) * (h - m), axis=-1, keepdims=True)
    return (h - m) / jnp.sqrt(v + 1e-5) * g + b


def _encode_body(x_ref, w0, b0, w1, b1, w2, b2, g, bt, o_ref):
    h = jnp.maximum(x_ref[...] @ w0[...] + b0[...], 0.0)
    h = jnp.maximum(h @ w1[...] + b1[...], 0.0)
    h = h @ w2[...] + b2[...]
    o_ref[...] = _ln(h, g[...], bt[...])


def _tc_encode(x, w0, b0, w1, b1, w2, b2, g, bt):
    return pl.pallas_call(
        _encode_body,
        grid=(GRID,),
        in_specs=[_rows(D), _full((D, HID)), _full((1, HID)),
                  _full((HID, HID)), _full((1, HID)),
                  _full((HID, LAT)), _full((1, LAT)),
                  _full((1, LAT)), _full((1, LAT))],
        out_specs=_rows(LAT),
        out_shape=jax.ShapeDtypeStruct((N, LAT), jnp.float32),
    )(x, w0, b0, w1, b1, w2, b2, g, bt)


def _mm_body(h_ref, wr, o_ref):
    o_ref[...] = h_ref[...] @ wr[...]


def _tc_mm(h, wr):
    return pl.pallas_call(
        _mm_body,
        grid=(GRID,),
        in_specs=[_rows(LAT), _full((LAT, HID))],
        out_specs=_rows(HID),
        out_shape=jax.ShapeDtypeStruct((N, HID), jnp.float32),
    )(h, wr)


def _comb_relu_body(sp_ref, dp_ref, hr_ref, wl, bl, o_ref):
    s = sp_ref[0] + sp_ref[1]
    deg = dp_ref[0][:, 0:1] + dp_ref[1][:, 0:1]
    aggr = s / jnp.maximum(deg, 1.0)
    o_ref[...] = jnp.maximum(aggr @ wl[...] + bl[...] + hr_ref[...], 0.0)


def _tc_comb_relu(sp, dp, hr, wl, bl):
    return pl.pallas_call(
        _comb_relu_body,
        grid=(GRID,),
        in_specs=[pl.BlockSpec((2, BR, LAT), lambda i: (0, i, 0)),
                  pl.BlockSpec((2, BR, 16), lambda i: (0, i, 0)),
                  _rows(HID), _full((LAT, HID)), _full((1, HID))],
        out_specs=_rows(HID),
        out_shape=jax.ShapeDtypeStruct((N, HID), jnp.float32),
    )(sp, dp, hr, wl, bl)


def _comb_ln_body(sp_ref, dp_ref, hr_ref, hres_ref, wl, bl, g, bt, o_ref):
    s = sp_ref[0] + sp_ref[1]
    deg = dp_ref[0][:, 0:1] + dp_ref[1][:, 0:1]
    aggr = s / jnp.maximum(deg, 1.0)
    h2 = aggr @ wl[...] + bl[...] + hr_ref[...]
    o_ref[...] = _ln(h2 + hres_ref[...], g[...], bt[...])


def _tc_comb_ln(sp, dp, hr, hres, wl, bl, g, bt):
    return pl.pallas_call(
        _comb_ln_body,
        grid=(GRID,),
        in_specs=[pl.BlockSpec((2, BR, HID), lambda i: (0, i, 0)),
                  pl.BlockSpec((2, BR, 16), lambda i: (0, i, 0)),
                  _rows(HID), _rows(LAT),
                  _full((HID, LAT)), _full((1, LAT)),
                  _full((1, LAT)), _full((1, LAT))],
        out_specs=_rows(LAT),
        out_shape=jax.ShapeDtypeStruct((N, LAT), jnp.float32),
    )(sp, dp, hr, hres, wl, bl, g, bt)


def _decode_body(h_ref, w0, b0, w1, b1, w2, b2, o_ref):
    o = jnp.maximum(h_ref[...] @ w0[...] + b0[...], 0.0)
    o = jnp.maximum(o @ w1[...] + b1[...], 0.0)
    o_ref[...] = o @ w2[...] + b2[...]


def _tc_decode(h, w0, b0, w1, b1, w2, b2):
    return pl.pallas_call(
        _decode_body,
        grid=(GRID,),
        in_specs=[_rows(LAT), _full((LAT, HID)), _full((1, HID)),
                  _full((HID, HID)), _full((1, HID)),
                  _full((HID, OUT)), _full((1, OUT))],
        out_specs=_rows(OUT),
        out_shape=jax.ShapeDtypeStruct((N, OUT), jnp.float32),
    )(h, w0, b0, w1, b1, w2, b2)


_NC = 2
_NS = 16
_NW = _NC * _NS
_EPW = E // _NW
_C = 80
_NCHUNK = _EPW // _C
_NPAD = 10240
_RPS = _NPAD // _NS

_sc_mesh = plsc.VectorSubcoreMesh(core_axis_name="c", subcore_axis_name="s")


_NBUF = 4


@functools.partial(
    pl.kernel,
    mesh=_sc_mesh,
    out_type=jax.ShapeDtypeStruct((_NC, _NPAD, LAT), jnp.float32),
    scratch_types=[
        pltpu.VMEM((_NBUF, _C), jnp.int32),
        pltpu.VMEM((_NBUF, _C), jnp.int32),
        pltpu.VMEM((_NBUF, _C, LAT), jnp.float32),
        pltpu.VMEM_SHARED((_NPAD, LAT), jnp.float32),
        pltpu.SemaphoreType.DMA((_NBUF,)),
        pltpu.SemaphoreType.DMA((_NBUF,)),
        pltpu.SemaphoreType.DMA((_NBUF,)),
    ],
)
def _sc_agg_kernel(h_hbm, src_hbm, dst_hbm, zeros_hbm, out_hbm,
                   sidx, didx, rows, acc, sem_i, sem_g, sem_s):
    cid = lax.axis_index("c")
    sid = lax.axis_index("s")
    wid = cid * _NS + sid
    pltpu.sync_copy(zeros_hbm.at[pl.ds(sid * _RPS, _RPS)],
                    acc.at[pl.ds(sid * _RPS, _RPS)])
    plsc.subcore_barrier()
    base = wid * _EPW

    def start_idx(off, b):
        pltpu.async_copy(src_hbm.at[pl.ds(off, _C)], sidx.at[b], sem_i.at[b])
        pltpu.async_copy(dst_hbm.at[pl.ds(off, _C)], didx.at[b], sem_i.at[b])

    def wait_idx(b):
        pltpu.make_async_copy(src_hbm.at[pl.ds(0, _C)], sidx.at[b], sem_i.at[b]).wait()
        pltpu.make_async_copy(src_hbm.at[pl.ds(0, _C)], didx.at[b], sem_i.at[b]).wait()

    def start_gather(b):
        pltpu.async_copy(h_hbm.at[sidx.at[b]], rows.at[b], sem_g.at[b])

    def wait_gather(b):
        pltpu.make_async_copy(h_hbm.at[pl.ds(0, _C)], rows.at[b], sem_g.at[b]).wait()

    def start_scatter(b):
        pltpu.async_copy(rows.at[b], acc.at[didx.at[b]], sem_s.at[b], add=True)

    def wait_scatter(b):
        pltpu.make_async_copy(h_hbm.at[pl.ds(0, _C)], rows.at[b], sem_s.at[b]).wait()

    start_idx(base, 0)
    wait_idx(0)
    start_gather(0)
    start_idx(base + _C, 1)
    wait_idx(1)
    start_gather(1)
    start_idx(base + 2 * _C, 2)
    wait_gather(0)
    start_scatter(0)
    wait_idx(2)
    start_gather(2)
    start_idx(base + 3 * _C, 3)
    wait_gather(1)
    start_scatter(1)

    def body(j, carry):
        c0 = 3 + 4 * j
        for i in range(4):
            b = (3 + i) % _NBUF
            pb = (2 + i) % _NBUF
            nb = (4 + i) % _NBUF
            wait_scatter(nb)
            start_idx(base + (c0 + i + 1) * _C, nb)
            wait_idx(b)
            start_gather(b)
            wait_gather(pb)
            start_scatter(pb)
        return carry

    lax.fori_loop(0, (_NCHUNK - 5) // 4, body, 0)

    wait_scatter(0)
    start_idx(base + 124 * _C, 0)
    wait_idx(3)
    start_gather(3)
    wait_gather(2)
    start_scatter(2)
    wait_idx(0)
    start_gather(0)
    wait_gather(3)
    start_scatter(3)
    wait_gather(0)
    start_scatter(0)
    wait_scatter(1)
    wait_scatter(2)
    wait_scatter(3)
    wait_scatter(0)

    plsc.subcore_barrier()
    pltpu.sync_copy(acc.at[pl.ds(sid * _RPS, _RPS)],
                    out_hbm.at[cid, pl.ds(sid * _RPS, _RPS)])


def _agg(h, src, dst, zeros):
    return _sc_agg_kernel(h, src, dst, zeros)


def _deg_partials(src, dst, zeros):
    ones = jnp.ones((N, LAT), jnp.float32)
    return _sc_agg_kernel(ones, src, dst, zeros)[:, :, :16]


def kernel(x, edge_index, eW0, eb0, eW1, eb1, eW2, eb2, eg, ebt,
           pW1l, pb1l, pW1r, pW2l, pb2l, pW2r, pg, pbt,
           dW0, db0, dW1, db1, dW2, db2):
    src = edge_index[0]
    dst = edge_index[1]
    r = lambda v: v.reshape(1, -1)

    zeros = jnp.zeros((_NPAD, LAT), jnp.float32)
    dp = _deg_partials(src, dst, zeros)
    h = _tc_encode(x, eW0, r(eb0), eW1, r(eb1), eW2, r(eb2), r(eg), r(ebt))
    for i in range(STEPS):
        sp = _agg(h, src, dst, zeros)
        hr = _tc_mm(h, pW1r[i])
        h1 = _tc_comb_relu(sp, dp, hr, pW1l[i], r(pb1l[i]))
        sp2 = _agg(h1, src, dst, zeros)
        hr2 = _tc_mm(h1, pW2r[i])
        h = _tc_comb_ln(sp2, dp, hr2, h, pW2l[i], r(pb2l[i]), r(pg[i]), r(pbt[i]))
    o = _tc_decode(h, dW0, r(db0), dW1, r(db1), dW2, r(db2))
    return o
